# baseline (speedup 1.0000x reference)
import jax
import jax.numpy as jnp
from jax.experimental import pallas as pl
from jax.experimental.pallas import tpu as pltpu

HEADS = 8
DIM_HEAD = 64
T_CHUNK = 512


def _mega_kernel(x_hbm, m_hbm, g_ref, bt_ref, wq_hbm, vis_hbm, wkv_hbm,
                 wo_hbm, o_hbm, wq_vm, wkv_vm, wo_vm, vis_vm, m_vm, x_vm,
                 o_vm, kbd_scr, wk_scr, vo_scr, cs2_scr, w3_scr, ocol_scr,
                 orow_scr, cones_scr, wsems, xsems, osems):
    B, T, C = x_hbm.shape
    V = vis_hbm.shape[1]
    inner = HEADS * DIM_HEAD
    HV = HEADS * V
    n_chunks_per_b = T // T_CHUNK
    n_chunks = B * n_chunks_per_b

    def chunk_xsrc(j):
        b, c = divmod(j, n_chunks_per_b)
        return x_hbm.at[b, pl.ds(c * T_CHUNK, T_CHUNK), :]

    c_wkv = pltpu.make_async_copy(wkv_hbm, wkv_vm, wsems.at[0])
    c_vis = pltpu.make_async_copy(vis_hbm, vis_vm, wsems.at[1])
    c_wq = pltpu.make_async_copy(wq_hbm, wq_vm, wsems.at[2])
    c_wo = pltpu.make_async_copy(wo_hbm, wo_vm, wsems.at[3])
    c_m = pltpu.make_async_copy(m_hbm, m_vm, wsems.at[4])
    c_x0 = pltpu.make_async_copy(chunk_xsrc(0), x_vm.at[0], xsems.at[0])
    c_x1 = pltpu.make_async_copy(chunk_xsrc(1), x_vm.at[1], xsems.at[1])
    c_wkv.start()
    c_vis.start()
    c_wq.start()
    c_x0.start()
    c_x1.start()
    c_wo.start()
    c_m.start()
    c_wkv.wait()
    c_vis.wait()
    c_wq.wait()
    c_wo.wait()
    c_m.wait()

    seg_c = jax.lax.broadcasted_iota(jnp.int32, (HV, HEADS), 0)
    hd_c = jax.lax.broadcasted_iota(jnp.int32, (HV, HEADS), 1)
    ocol_scr[...] = (seg_c // V == hd_c).astype(jnp.float32)
    seg_r = jax.lax.broadcasted_iota(jnp.int32, (HEADS, HV), 1)
    hd_r = jax.lax.broadcasted_iota(jnp.int32, (HEADS, HV), 0)
    orow_scr[...] = (seg_r // V == hd_r).astype(jnp.float32)
    cones_scr[...] = jnp.full((C, 8), 1.0 / C, jnp.float32)
    tg = jnp.transpose(g_ref[...], (1, 0))

    def prep(b):
        vis = vis_vm[b]
        kv = jnp.dot(vis, wkv_vm[...], preferred_element_type=jnp.float32)
        scale = jnp.float32(DIM_HEAD ** -0.5)
        kbd_scr[...] = jnp.zeros((inner, HV), jnp.float32)
        for h in range(HEADS):
            kh = kv[:, h * DIM_HEAD:(h + 1) * DIM_HEAD]
            kbd_scr[h * DIM_HEAD:(h + 1) * DIM_HEAD,
                    h * V:(h + 1) * V] = kh.T * scale
        kbd = kbd_scr[...]
        gwq = wq_vm[...] * tg
        wk_scr[...] = jnp.dot(gwq, kbd, preferred_element_type=jnp.float32)
        csum = jnp.dot(jnp.full((8, C), 1.0, jnp.float32), gwq,
                       preferred_element_type=jnp.float32)
        cs2_scr[...] = jnp.dot(csum, kbd, preferred_element_type=jnp.float32)
        bt8 = jnp.concatenate([bt_ref[...]] * 8, axis=0)
        w3_scr[...] = jnp.dot(
            jnp.dot(bt8, wq_vm[...], preferred_element_type=jnp.float32),
            kbd, preferred_element_type=jnp.float32)
        for h in range(HEADS):
            vh = kv[:, inner + h * DIM_HEAD:inner + (h + 1) * DIM_HEAD]
            wo_h = wo_vm[h * DIM_HEAD:(h + 1) * DIM_HEAD, :]
            vo_scr[h * V:(h + 1) * V, :] = jnp.dot(
                vh, wo_h, preferred_element_type=jnp.float32)

    def compute_chunk(j):
        b, c = divmod(j, n_chunks_per_b)
        s = j % 2
        xb = x_vm[s]
        sq = xb * xb
        cones = cones_scr[...]
        mu8 = jnp.dot(xb, cones, preferred_element_type=jnp.float32)
        m28 = jnp.dot(sq, cones, preferred_element_type=jnp.float32)
        s8 = jax.lax.rsqrt(m28 - mu8 * mu8 + 1e-5)
        orow = orow_scr[...]
        mu_f = jnp.dot(mu8, orow, preferred_element_type=jnp.float32)
        s_f = jnp.dot(s8, orow, preferred_element_type=jnp.float32)
        z = jnp.dot(xb, wk_scr[...], preferred_element_type=jnp.float32)
        mt = m_vm[b][:, c * T_CHUNK:(c + 1) * T_CHUNK].T
        neg = jnp.where(mt != 0, 0.0, -10000.0).astype(jnp.float32)
        neg8 = jnp.concatenate([neg] * HEADS, axis=-1)
        e8 = jnp.exp((z - mu_f * cs2_scr[0:1, :]) * s_f + neg8
                     + w3_scr[0:1, :])
        d8 = jnp.dot(e8, ocol_scr[...], preferred_element_type=jnp.float32)
        r8 = 1.0 / jnp.maximum(d8, 1e-30)
        rfull = jnp.dot(r8, orow, preferred_element_type=jnp.float32)
        p = e8 * rfull
        o_vm[s] = jnp.dot(p, vo_scr[...], preferred_element_type=jnp.float32)

    out_copies = [None] * n_chunks
    for j in range(n_chunks):
        b, c = divmod(j, n_chunks_per_b)
        s = j % 2
        if c == 0:
            prep(b)
        pltpu.make_async_copy(chunk_xsrc(j), x_vm.at[s], xsems.at[s]).wait()
        if j >= 2:
            out_copies[j - 2].wait()
        compute_chunk(j)
        oc = pltpu.make_async_copy(
            o_vm.at[s], o_hbm.at[b, pl.ds(c * T_CHUNK, T_CHUNK), :],
            osems.at[s])
        oc.start()
        out_copies[j] = oc
        if j + 2 < n_chunks:
            pltpu.make_async_copy(chunk_xsrc(j + 2), x_vm.at[s],
                                  xsems.at[s]).start()
    out_copies[n_chunks - 2].wait()
    out_copies[n_chunks - 1].wait()


def kernel(x, vision, attention_mask, ln_g, ln_b, Wq, Wkv, Wo):
    B, T, C = x.shape
    V = vision.shape[1]
    inner = HEADS * DIM_HEAD
    HV = HEADS * V
    g2 = ln_g.reshape(1, C)
    b2 = ln_b.reshape(1, C)
    any_spec = pl.BlockSpec(memory_space=pl.ANY)
    vmem = pltpu.VMEM
    return pl.pallas_call(
        _mega_kernel,
        in_specs=[
            any_spec,
            any_spec,
            pl.BlockSpec((1, C), lambda: (0, 0)),
            pl.BlockSpec((1, C), lambda: (0, 0)),
            any_spec,
            any_spec,
            any_spec,
            any_spec,
        ],
        out_specs=any_spec,
        out_shape=jax.ShapeDtypeStruct((B, T, C), jnp.float32),
        scratch_shapes=[
            vmem((C, inner), jnp.float32),
            vmem((C, 2 * inner), jnp.float32),
            vmem((inner, C), jnp.float32),
            vmem((B, V, C), jnp.float32),
            vmem((B, V, T), jnp.int32),
            vmem((2, T_CHUNK, C), jnp.float32),
            vmem((2, T_CHUNK, C), jnp.float32),
            vmem((inner, HV), jnp.float32),
            vmem((C, HV), jnp.float32),
            vmem((HV, C), jnp.float32),
            vmem((8, HV), jnp.float32),
            vmem((8, HV), jnp.float32),
            vmem((HV, 8), jnp.float32),
            vmem((8, HV), jnp.float32),
            vmem((C, 8), jnp.float32),
            pltpu.SemaphoreType.DMA((8,)),
            pltpu.SemaphoreType.DMA((2,)),
            pltpu.SemaphoreType.DMA((2,)),
        ],
    )(x, attention_mask.astype(jnp.int32), g2, b2, Wq, vision, Wkv, Wo)


# speedup vs baseline: 1.2163x; 1.2163x over previous
import jax
import jax.numpy as jnp
from jax.experimental import pallas as pl
from jax.experimental.pallas import tpu as pltpu

HEADS = 8
DIM_HEAD = 64
T_CHUNK = 1024


def _mega_kernel(x_hbm, m_hbm, g_ref, bt_ref, wq_hbm, vis_hbm, wkv_hbm,
                 wo_hbm, o_hbm, wq_vm, wkv_vm, wo_vm, vis_vm, m_vm, x_vm,
                 o_vm, kbd_scr, vo_scr, ocol_scr, orow_scr, wsems, xsems,
                 osems):
    B, T, C = x_hbm.shape
    V = vis_hbm.shape[1]
    inner = HEADS * DIM_HEAD
    HV = HEADS * V
    n_chunks_per_b = T // T_CHUNK
    n_chunks = B * n_chunks_per_b

    def chunk_xsrc(j):
        b, c = divmod(j, n_chunks_per_b)
        return x_hbm.at[b, pl.ds(c * T_CHUNK, T_CHUNK), :]

    c_wkv = pltpu.make_async_copy(wkv_hbm, wkv_vm, wsems.at[0])
    c_vis = pltpu.make_async_copy(vis_hbm, vis_vm, wsems.at[1])
    c_wq = pltpu.make_async_copy(wq_hbm, wq_vm, wsems.at[2])
    c_wo = pltpu.make_async_copy(wo_hbm, wo_vm, wsems.at[3])
    c_m = pltpu.make_async_copy(m_hbm, m_vm, wsems.at[4])
    c_x0 = pltpu.make_async_copy(chunk_xsrc(0), x_vm.at[0], xsems.at[0])
    c_x1 = pltpu.make_async_copy(chunk_xsrc(1), x_vm.at[1], xsems.at[1])
    c_wkv.start()
    c_vis.start()
    c_wq.start()
    c_x0.start()
    c_x1.start()
    c_wo.start()
    c_m.start()
    c_wkv.wait()
    c_vis.wait()
    c_wq.wait()
    c_wo.wait()
    c_m.wait()

    seg_c = jax.lax.broadcasted_iota(jnp.int32, (HV, HEADS), 0)
    hd_c = jax.lax.broadcasted_iota(jnp.int32, (HV, HEADS), 1)
    ocol_scr[...] = (seg_c // V == hd_c).astype(jnp.float32)
    seg_r = jax.lax.broadcasted_iota(jnp.int32, (HEADS, HV), 1)
    hd_r = jax.lax.broadcasted_iota(jnp.int32, (HEADS, HV), 0)
    orow_scr[...] = (seg_r // V == hd_r).astype(jnp.float32)

    def prep(b):
        vis = vis_vm[b]
        kv = jnp.dot(vis, wkv_vm[...], preferred_element_type=jnp.float32)
        scale = jnp.float32(DIM_HEAD ** -0.5)
        kbd_scr[...] = jnp.zeros((inner, HV), jnp.float32)
        for h in range(HEADS):
            kh = kv[:, h * DIM_HEAD:(h + 1) * DIM_HEAD]
            kbd_scr[h * DIM_HEAD:(h + 1) * DIM_HEAD,
                    h * V:(h + 1) * V] = kh.T * scale
        for h in range(HEADS):
            vh = kv[:, inner + h * DIM_HEAD:inner + (h + 1) * DIM_HEAD]
            wo_h = wo_vm[h * DIM_HEAD:(h + 1) * DIM_HEAD, :]
            vo_scr[h * V:(h + 1) * V, :] = jnp.dot(
                vh, wo_h, preferred_element_type=jnp.float32)

    def compute_chunk(j):
        b, c = divmod(j, n_chunks_per_b)
        s = j % 2
        xb = x_vm[s]
        mu = jnp.mean(xb, axis=-1, keepdims=True)
        var = jnp.mean((xb - mu) ** 2, axis=-1, keepdims=True)
        xn = (xb - mu) * jax.lax.rsqrt(var + 1e-5) * g_ref[0] + bt_ref[0]
        q = jnp.dot(xn, wq_vm[...], preferred_element_type=jnp.float32)
        mt = m_vm[b][:, c * T_CHUNK:(c + 1) * T_CHUNK].T
        neg = jnp.where(mt != 0, 0.0, -10000.0).astype(jnp.float32)
        neg8 = jnp.concatenate([neg] * HEADS, axis=-1)
        sim8 = jnp.dot(q, kbd_scr[...], preferred_element_type=jnp.float32)
        e8 = jnp.exp(sim8 + neg8)
        d8 = jnp.dot(e8, ocol_scr[...], preferred_element_type=jnp.float32)
        r8 = 1.0 / jnp.maximum(d8, 1e-30)
        rfull = jnp.dot(r8, orow_scr[...],
                        preferred_element_type=jnp.float32)
        p = e8 * rfull
        o_vm[s] = jnp.dot(p, vo_scr[...], preferred_element_type=jnp.float32)

    out_copies = [None] * n_chunks
    for j in range(n_chunks):
        b, c = divmod(j, n_chunks_per_b)
        s = j % 2
        if c == 0:
            prep(b)
        pltpu.make_async_copy(chunk_xsrc(j), x_vm.at[s], xsems.at[s]).wait()
        if j >= 2:
            out_copies[j - 2].wait()
        compute_chunk(j)
        oc = pltpu.make_async_copy(
            o_vm.at[s], o_hbm.at[b, pl.ds(c * T_CHUNK, T_CHUNK), :],
            osems.at[s])
        oc.start()
        out_copies[j] = oc
        if j + 2 < n_chunks:
            pltpu.make_async_copy(chunk_xsrc(j + 2), x_vm.at[s],
                                  xsems.at[s]).start()
    out_copies[n_chunks - 2].wait()
    out_copies[n_chunks - 1].wait()


def kernel(x, vision, attention_mask, ln_g, ln_b, Wq, Wkv, Wo):
    B, T, C = x.shape
    V = vision.shape[1]
    inner = HEADS * DIM_HEAD
    HV = HEADS * V
    g2 = ln_g.reshape(1, C)
    b2 = ln_b.reshape(1, C)
    any_spec = pl.BlockSpec(memory_space=pl.ANY)
    vmem = pltpu.VMEM
    return pl.pallas_call(
        _mega_kernel,
        in_specs=[
            any_spec,
            any_spec,
            pl.BlockSpec((1, C), lambda: (0, 0)),
            pl.BlockSpec((1, C), lambda: (0, 0)),
            any_spec,
            any_spec,
            any_spec,
            any_spec,
        ],
        out_specs=any_spec,
        out_shape=jax.ShapeDtypeStruct((B, T, C), jnp.float32),
        scratch_shapes=[
            vmem((C, inner), jnp.float32),
            vmem((C, 2 * inner), jnp.float32),
            vmem((inner, C), jnp.float32),
            vmem((B, V, C), jnp.float32),
            vmem((B, V, T), jnp.int32),
            vmem((2, T_CHUNK, C), jnp.float32),
            vmem((2, T_CHUNK, C), jnp.float32),
            vmem((inner, HV), jnp.float32),
            vmem((HV, C), jnp.float32),
            vmem((HV, 8), jnp.float32),
            vmem((8, HV), jnp.float32),
            pltpu.SemaphoreType.DMA((8,)),
            pltpu.SemaphoreType.DMA((2,)),
            pltpu.SemaphoreType.DMA((2,)),
        ],
    )(x, attention_mask.astype(jnp.int32), g2, b2, Wq, vision, Wkv, Wo)


# speedup vs baseline: 1.2508x; 1.0283x over previous
import functools

import jax
import jax.numpy as jnp
from jax.experimental import pallas as pl
from jax.experimental.pallas import tpu as pltpu

HEADS = 8
DIM_HEAD = 64
T_TILE = 1024


def _fused_kernel(x_ref, m_ref, g_ref, bt_ref, wq_hbm, vis_hbm, wkv_hbm,
                  wo_hbm, o_ref, wq_vm, wkv_vm, wo_vm, vis_vm, kbd_scr,
                  vo_scr, ocol_scr, orow_scr, wsems, *, inner, V):
    b = pl.program_id(0)
    t = pl.program_id(1)

    @pl.when(jnp.logical_and(b == 0, t == 0))
    def _load_weights():
        c_wkv = pltpu.make_async_copy(wkv_hbm, wkv_vm, wsems.at[0])
        c_vis = pltpu.make_async_copy(vis_hbm, vis_vm, wsems.at[1])
        c_wq = pltpu.make_async_copy(wq_hbm, wq_vm, wsems.at[2])
        c_wo = pltpu.make_async_copy(wo_hbm, wo_vm, wsems.at[3])
        c_wkv.start()
        c_vis.start()
        c_wq.start()
        c_wo.start()
        c_wkv.wait()
        c_vis.wait()
        c_wq.wait()
        c_wo.wait()

    @pl.when(t == 0)
    def _prep():
        vis = vis_vm[b]
        kv = jnp.dot(vis, wkv_vm[...], preferred_element_type=jnp.float32)
        scale = jnp.float32(DIM_HEAD ** -0.5)
        kbd_scr[...] = jnp.zeros((inner, HEADS * V), jnp.float32)
        for h in range(HEADS):
            kh = kv[:, h * DIM_HEAD:(h + 1) * DIM_HEAD]
            kbd_scr[h * DIM_HEAD:(h + 1) * DIM_HEAD,
                    h * V:(h + 1) * V] = kh.T * scale
            vh = kv[:, inner + h * DIM_HEAD:inner + (h + 1) * DIM_HEAD]
            wo_h = wo_vm[h * DIM_HEAD:(h + 1) * DIM_HEAD, :]
            vo_scr[h * V:(h + 1) * V, :] = jnp.dot(
                vh, wo_h, preferred_element_type=jnp.float32)
        seg_c = jax.lax.broadcasted_iota(jnp.int32, (HEADS * V, HEADS), 0)
        hd_c = jax.lax.broadcasted_iota(jnp.int32, (HEADS * V, HEADS), 1)
        ocol_scr[...] = (seg_c // V == hd_c).astype(jnp.float32)
        seg_r = jax.lax.broadcasted_iota(jnp.int32, (HEADS, HEADS * V), 1)
        hd_r = jax.lax.broadcasted_iota(jnp.int32, (HEADS, HEADS * V), 0)
        orow_scr[...] = (seg_r // V == hd_r).astype(jnp.float32)

    xb = x_ref[0]
    mu = jnp.mean(xb, axis=-1, keepdims=True)
    var = jnp.mean((xb - mu) ** 2, axis=-1, keepdims=True)
    xn = (xb - mu) * jax.lax.rsqrt(var + 1e-5) * g_ref[0] + bt_ref[0]
    q = jnp.dot(xn, wq_vm[...], preferred_element_type=jnp.float32)

    mt = m_ref[0].T
    neg = jnp.where(mt != 0, 0.0, -10000.0).astype(jnp.float32)
    neg8 = jnp.concatenate([neg] * HEADS, axis=-1)

    sim8 = jnp.dot(q, kbd_scr[...], preferred_element_type=jnp.float32)
    e8 = jnp.exp(sim8 + neg8)
    d8 = jnp.dot(e8, ocol_scr[...], preferred_element_type=jnp.float32)
    r8 = 1.0 / jnp.maximum(d8, 1e-30)
    rfull = jnp.dot(r8, orow_scr[...], preferred_element_type=jnp.float32)
    p = e8 * rfull
    o_ref[0] = jnp.dot(p, vo_scr[...], preferred_element_type=jnp.float32)


def kernel(x, vision, attention_mask, ln_g, ln_b, Wq, Wkv, Wo):
    B, T, C = x.shape
    V = vision.shape[1]
    inner = HEADS * DIM_HEAD
    g2 = ln_g.reshape(1, C)
    b2 = ln_b.reshape(1, C)
    grid = (B, T // T_TILE)
    any_spec = pl.BlockSpec(memory_space=pl.ANY)
    return pl.pallas_call(
        functools.partial(_fused_kernel, inner=inner, V=V),
        grid=grid,
        in_specs=[
            pl.BlockSpec((1, T_TILE, C), lambda b, t: (b, t, 0)),
            pl.BlockSpec((1, V, T_TILE), lambda b, t: (b, 0, t)),
            pl.BlockSpec((1, C), lambda b, t: (0, 0)),
            pl.BlockSpec((1, C), lambda b, t: (0, 0)),
            any_spec,
            any_spec,
            any_spec,
            any_spec,
        ],
        out_specs=pl.BlockSpec((1, T_TILE, C), lambda b, t: (b, t, 0)),
        out_shape=jax.ShapeDtypeStruct((B, T, C), jnp.float32),
        scratch_shapes=[
            pltpu.VMEM((C, inner), jnp.float32),
            pltpu.VMEM((C, 2 * inner), jnp.float32),
            pltpu.VMEM((inner, C), jnp.float32),
            pltpu.VMEM((B, V, C), jnp.float32),
            pltpu.VMEM((inner, HEADS * V), jnp.float32),
            pltpu.VMEM((HEADS * V, C), jnp.float32),
            pltpu.VMEM((HEADS * V, HEADS), jnp.float32),
            pltpu.VMEM((HEADS, HEADS * V), jnp.float32),
            pltpu.SemaphoreType.DMA((4,)),
        ],
    )(x, attention_mask.astype(jnp.int32), g2, b2, Wq, vision, Wkv, Wo)


# speedup vs baseline: 1.3918x; 1.1127x over previous
import functools

import jax
import jax.numpy as jnp
from jax.experimental import pallas as pl
from jax.experimental.pallas import tpu as pltpu

HEADS = 8
DIM_HEAD = 64
T_TILE = 1024


def _fused_kernel(x_ref, m_ref, g_ref, bt_ref, wq_ref, vis_ref, wkv_ref,
                  wo_ref, o_ref, kbd_scr, vo_scr, ocol_scr, orow_scr, *,
                  inner, V):
    t = pl.program_id(1)

    @pl.when(t == 0)
    def _prep():
        vis = vis_ref[0]
        kv = jnp.dot(vis, wkv_ref[...], preferred_element_type=jnp.float32)
        scale = jnp.float32(DIM_HEAD ** -0.5)
        kbd_scr[...] = jnp.zeros((inner, HEADS * V), jnp.float32)
        for h in range(HEADS):
            kh = kv[:, h * DIM_HEAD:(h + 1) * DIM_HEAD]
            kbd_scr[h * DIM_HEAD:(h + 1) * DIM_HEAD,
                    h * V:(h + 1) * V] = kh.T * scale
            vh = kv[:, inner + h * DIM_HEAD:inner + (h + 1) * DIM_HEAD]
            wo_h = wo_ref[h * DIM_HEAD:(h + 1) * DIM_HEAD, :]
            vo_scr[h * V:(h + 1) * V, :] = jnp.dot(
                vh, wo_h, preferred_element_type=jnp.float32)
        seg_c = jax.lax.broadcasted_iota(jnp.int32, (HEADS * V, HEADS), 0)
        hd_c = jax.lax.broadcasted_iota(jnp.int32, (HEADS * V, HEADS), 1)
        ocol_scr[...] = (seg_c // V == hd_c).astype(jnp.float32)
        seg_r = jax.lax.broadcasted_iota(jnp.int32, (HEADS, HEADS * V), 1)
        hd_r = jax.lax.broadcasted_iota(jnp.int32, (HEADS, HEADS * V), 0)
        orow_scr[...] = (seg_r // V == hd_r).astype(jnp.float32)

    xb = x_ref[0]
    mu = jnp.mean(xb, axis=-1, keepdims=True)
    var = jnp.mean((xb - mu) ** 2, axis=-1, keepdims=True)
    xn = (xb - mu) * jax.lax.rsqrt(var + 1e-5) * g_ref[0] + bt_ref[0]
    q = jnp.dot(xn, wq_ref[...], preferred_element_type=jnp.float32)

    mt = m_ref[0].T
    neg = jnp.where(mt != 0, 0.0, -10000.0).astype(jnp.float32)
    neg8 = jnp.concatenate([neg] * HEADS, axis=-1)

    sim8 = jnp.dot(q, kbd_scr[...], preferred_element_type=jnp.float32)
    e8 = jnp.exp(sim8 + neg8)
    d8 = jnp.dot(e8, ocol_scr[...], preferred_element_type=jnp.float32)
    r8 = 1.0 / jnp.maximum(d8, 1e-30)
    rfull = jnp.dot(r8, orow_scr[...], preferred_element_type=jnp.float32)
    p = e8 * rfull
    o_ref[0] = jnp.dot(p, vo_scr[...], preferred_element_type=jnp.float32)


def kernel(x, vision, attention_mask, ln_g, ln_b, Wq, Wkv, Wo):
    B, T, C = x.shape
    V = vision.shape[1]
    inner = HEADS * DIM_HEAD
    g2 = ln_g.reshape(1, C)
    b2 = ln_b.reshape(1, C)
    grid = (B, T // T_TILE)
    return pl.pallas_call(
        functools.partial(_fused_kernel, inner=inner, V=V),
        grid=grid,
        in_specs=[
            pl.BlockSpec((1, T_TILE, C), lambda b, t: (b, t, 0)),
            pl.BlockSpec((1, V, T_TILE), lambda b, t: (b, 0, t)),
            pl.BlockSpec((1, C), lambda b, t: (0, 0)),
            pl.BlockSpec((1, C), lambda b, t: (0, 0)),
            pl.BlockSpec((C, inner), lambda b, t: (0, 0)),
            pl.BlockSpec((1, V, C), lambda b, t: (b, 0, 0)),
            pl.BlockSpec((C, 2 * inner), lambda b, t: (0, 0)),
            pl.BlockSpec((inner, C), lambda b, t: (0, 0)),
        ],
        out_specs=pl.BlockSpec((1, T_TILE, C), lambda b, t: (b, t, 0)),
        out_shape=jax.ShapeDtypeStruct((B, T, C), jnp.float32),
        compiler_params=pltpu.CompilerParams(
            dimension_semantics=("parallel", "arbitrary")),
        scratch_shapes=[
            pltpu.VMEM((inner, HEADS * V), jnp.float32),
            pltpu.VMEM((HEADS * V, C), jnp.float32),
            pltpu.VMEM((HEADS * V, HEADS), jnp.float32),
            pltpu.VMEM((HEADS, HEADS * V), jnp.float32),
        ],
    )(x, attention_mask.astype(jnp.int32), g2, b2, Wq, vision, Wkv, Wo)
